# baseline (speedup 1.0000x reference)
import functools
import jax, jax.numpy as jnp
from jax import lax
from jax.experimental import pallas as pl
from jax.experimental.pallas import tpu as pltpu

_VOCAB = 100000
_EMB = 64
_CTX = 10
_HID = 128
_MAINW = 99840
_TAILW = _VOCAB - _MAINW
_TPAD = 256
_OUTW = _MAINW + _TPAD
_WIDTHS = [12800] * 7 + [10240]


def _body(idx_ref, emb_ref, W1_ref, b1_ref, W2_ref, b2_ref, w2t_ref, b2t_ref,
          out_ref, xg_ref, buf_ref, sems_ref, gsem_ref):
    gathers = [
        pltpu.make_async_copy(
            emb_ref.at[pl.ds(idx_ref[r], 1), :],
            xg_ref.at[pl.ds(r, 1), :],
            gsem_ref,
        )
        for r in range(2 * _CTX)
    ]
    for g in gathers:
        g.start()

    cps = [
        pltpu.make_async_copy(
            W2_ref.at[pl.ds(64 * i, 64), :],
            buf_ref.at[pl.ds(64 * i, 64), :],
            sems_ref.at[i],
        )
        for i in range(2)
    ]
    for cp in cps:
        cp.start()

    for g in gathers:
        g.wait()
    h = b1_ref[...]
    for r in range(2 * _CTX):
        h = h + jnp.dot(xg_ref[pl.ds(r, 1), :], W1_ref[r],
                        preferred_element_type=jnp.float32)
    h16 = jnp.maximum(h, 0.0).astype(jnp.bfloat16)

    zt = jnp.dot(h16, w2t_ref[...].astype(jnp.bfloat16),
                 preferred_element_type=jnp.float32) + b2t_ref[...]
    out_ref[:, pl.ds(_MAINW, _TPAD)] = zt
    m = jnp.max(zt)
    s = jnp.sum(jnp.exp(zt - m))

    for cp in cps:
        cp.wait()

    off = 0
    for w in _WIDTHS:
        z = jnp.dot(h16, buf_ref[:, pl.ds(off, w)].astype(jnp.bfloat16),
                    preferred_element_type=jnp.float32)
        z = z + b2_ref[:, pl.ds(off, w)]
        out_ref[:, pl.ds(off, w)] = z
        m_new = jnp.maximum(m, jnp.max(z))
        s = s * jnp.exp(m - m_new) + jnp.sum(jnp.exp(z - m_new))
        m = m_new
        off += w

    lse = m + jnp.log(s)
    off = 0
    for w in _WIDTHS + [_TPAD]:
        sl = pl.ds(off, w)
        out_ref[:, sl] = out_ref[:, sl] - lse
        off += w


def kernel(inputs, emb, W1, b1, W2, b2):
    idx = jnp.asarray(inputs, jnp.int32)
    W1r = W1.reshape(2 * _CTX, _EMB, _HID)
    b1r = b1.reshape(1, _HID)
    b2r = b2.reshape(1, _VOCAB)
    w2t = jnp.pad(lax.slice(W2, (0, _MAINW), (_HID, _VOCAB)),
                  ((0, 0), (0, _TPAD - _TAILW)))
    b2t = jnp.pad(lax.slice(b2r, (0, _MAINW), (1, _VOCAB)),
                  ((0, 0), (0, _TPAD - _TAILW)), constant_values=-3.0e38)

    grid_spec = pltpu.PrefetchScalarGridSpec(
        num_scalar_prefetch=1,
        grid=(1,),
        in_specs=[
            pl.BlockSpec(memory_space=pltpu.HBM),
            pl.BlockSpec((2 * _CTX, _EMB, _HID), lambda i, idx_ref: (0, 0, 0)),
            pl.BlockSpec((1, _HID), lambda i, idx_ref: (0, 0)),
            pl.BlockSpec(memory_space=pltpu.HBM),
            pl.BlockSpec((1, _VOCAB), lambda i, idx_ref: (0, 0)),
            pl.BlockSpec((_HID, _TPAD), lambda i, idx_ref: (0, 0)),
            pl.BlockSpec((1, _TPAD), lambda i, idx_ref: (0, 0)),
        ],
        out_specs=pl.BlockSpec((1, _OUTW), lambda i, idx_ref: (0, 0)),
        scratch_shapes=[
            pltpu.VMEM((2 * _CTX, _EMB), jnp.float32),
            pltpu.VMEM((_HID, _VOCAB), jnp.float32),
            pltpu.SemaphoreType.DMA((2,)),
            pltpu.SemaphoreType.DMA,
        ],
    )

    out = pl.pallas_call(
        _body,
        grid_spec=grid_spec,
        out_shape=jax.ShapeDtypeStruct((1, _OUTW), jnp.float32),
        compiler_params=pltpu.CompilerParams(
            vmem_limit_bytes=120 * 1024 * 1024,
        ),
    )(idx, emb, W1r, b1r, W2, b2r, w2t, b2t)
    return out[:, :_VOCAB]


# speedup vs baseline: 1.0095x; 1.0095x over previous
import functools
import jax, jax.numpy as jnp
from jax import lax
from jax.experimental import pallas as pl
from jax.experimental.pallas import tpu as pltpu

_VOCAB = 100000
_EMB = 64
_CTX = 10
_HID = 128
_MAINW = 99840
_TAILW = _VOCAB - _MAINW
_TPAD = 256
_OUTW = _MAINW + _TPAD
_WIDTHS = [12800] * 7 + [10240]


def _body(idx_ref, emb_ref, W1_ref, b1_ref, W2_ref, b2_ref, w2t_ref, b2t_ref,
          out_ref, xg_ref, buf_ref, sems_ref, gsem_ref):
    cps = [
        pltpu.make_async_copy(
            W2_ref.at[pl.ds(64 * i, 64), :],
            buf_ref.at[pl.ds(64 * i, 64), :],
            sems_ref.at[i],
        )
        for i in range(2)
    ]
    for cp in cps:
        cp.start()

    gathers = [
        pltpu.make_async_copy(
            emb_ref.at[pl.ds(idx_ref[r], 1), :],
            xg_ref.at[pl.ds(r, 1), :],
            gsem_ref,
        )
        for r in range(2 * _CTX)
    ]
    for g in gathers:
        g.start()
    for g in gathers:
        g.wait()

    h = b1_ref[...]
    for r in range(2 * _CTX):
        h = h + jnp.dot(xg_ref[pl.ds(r, 1), :], W1_ref[r],
                        preferred_element_type=jnp.float32)
    h16 = jnp.maximum(h, 0.0).astype(jnp.bfloat16)

    zt = jnp.dot(h16, w2t_ref[...].astype(jnp.bfloat16),
                 preferred_element_type=jnp.float32) + b2t_ref[...]
    out_ref[:, pl.ds(_MAINW, _TPAD)] = zt
    m = jnp.max(zt)
    s = jnp.sum(jnp.exp(zt - m))

    for cp in cps:
        cp.wait()

    off = 0
    for w in _WIDTHS:
        z = jnp.dot(h16, buf_ref[:, pl.ds(off, w)].astype(jnp.bfloat16),
                    preferred_element_type=jnp.float32)
        z = z + b2_ref[:, pl.ds(off, w)]
        out_ref[:, pl.ds(off, w)] = z
        m_new = jnp.maximum(m, jnp.max(z))
        s = s * jnp.exp(m - m_new) + jnp.sum(jnp.exp(z - m_new))
        m = m_new
        off += w

    lse = m + jnp.log(s)
    off = 0
    for w in _WIDTHS + [_TPAD]:
        sl = pl.ds(off, w)
        out_ref[:, sl] = out_ref[:, sl] - lse
        off += w


def kernel(inputs, emb, W1, b1, W2, b2):
    idx = jnp.asarray(inputs, jnp.int32)
    W1r = W1.reshape(2 * _CTX, _EMB, _HID)
    b1r = b1.reshape(1, _HID)
    b2r = b2.reshape(1, _VOCAB)
    w2t = jnp.pad(lax.slice(W2, (0, _MAINW), (_HID, _VOCAB)),
                  ((0, 0), (0, _TPAD - _TAILW)))
    b2t = jnp.pad(lax.slice(b2r, (0, _MAINW), (1, _VOCAB)),
                  ((0, 0), (0, _TPAD - _TAILW)), constant_values=-3.0e38)

    out = pl.pallas_call(
        _body,
        grid=(1,),
        in_specs=[
            pl.BlockSpec(memory_space=pltpu.SMEM),
            pl.BlockSpec(memory_space=pltpu.HBM),
            pl.BlockSpec((2 * _CTX, _EMB, _HID), lambda i: (0, 0, 0)),
            pl.BlockSpec((1, _HID), lambda i: (0, 0)),
            pl.BlockSpec(memory_space=pltpu.HBM),
            pl.BlockSpec((1, _VOCAB), lambda i: (0, 0)),
            pl.BlockSpec((_HID, _TPAD), lambda i: (0, 0)),
            pl.BlockSpec((1, _TPAD), lambda i: (0, 0)),
        ],
        out_specs=pl.BlockSpec((1, _OUTW), lambda i: (0, 0)),
        out_shape=jax.ShapeDtypeStruct((1, _OUTW), jnp.float32),
        scratch_shapes=[
            pltpu.VMEM((2 * _CTX, _EMB), jnp.float32),
            pltpu.VMEM((_HID, _VOCAB), jnp.float32),
            pltpu.SemaphoreType.DMA((2,)),
            pltpu.SemaphoreType.DMA,
        ],
        compiler_params=pltpu.CompilerParams(
            vmem_limit_bytes=120 * 1024 * 1024,
        ),
    )(idx, emb, W1r, b1r, W2, b2r, w2t, b2t)
    return out[:, :_VOCAB]


# speedup vs baseline: 1.0125x; 1.0030x over previous
import functools
import jax, jax.numpy as jnp
from jax import lax
from jax.experimental import pallas as pl
from jax.experimental.pallas import tpu as pltpu

_VOCAB = 100000
_EMB = 64
_CTX = 10
_HID = 128
_BC = 4096
_NCH = _VOCAB // _BC
_TAIL = _VOCAB - _NCH * _BC
_TPAD = 2048
_VPAD = _NCH * _BC + _TPAD
_NBUF = 4


def _body(idx_ref, emb_ref, W1_ref, b1_ref, W2_ref, b2_ref, w2t_ref, b2t_ref,
          out_ref, xg_ref, bufs_ref, sems_ref, gsem_ref):
    def w2_copy(c, b):
        return pltpu.make_async_copy(
            W2_ref.at[:, pl.ds(c * _BC, _BC)],
            bufs_ref.at[b],
            sems_ref.at[b],
        )

    for b in range(_NBUF):
        w2_copy(b, b).start()
    gathers = [
        pltpu.make_async_copy(
            emb_ref.at[pl.ds(idx_ref[r], 1), :],
            xg_ref.at[pl.ds(r, 1), :],
            gsem_ref,
        )
        for r in range(2 * _CTX)
    ]
    for g in gathers:
        g.start()
    for g in gathers:
        g.wait()

    h = b1_ref[...]
    for r in range(2 * _CTX):
        h = h + jnp.dot(xg_ref[pl.ds(r, 1), :], W1_ref[r],
                        preferred_element_type=jnp.float32)
    h16 = jnp.maximum(h, 0.0).astype(jnp.bfloat16)

    m = jnp.float32(-3.0e38)
    s = jnp.float32(0.0)
    for c in range(_NCH):
        b = c % _NBUF
        w2_copy(c, b).wait()
        z = jnp.dot(h16, bufs_ref[b].astype(jnp.bfloat16),
                    preferred_element_type=jnp.float32)
        if c + _NBUF < _NCH:
            w2_copy(c + _NBUF, b).start()
        z = z + b2_ref[:, pl.ds(c * _BC, _BC)]
        out_ref[:, pl.ds(c * _BC, _BC)] = z
        m_new = jnp.maximum(m, jnp.max(z))
        s = s * jnp.exp(m - m_new) + jnp.sum(jnp.exp(z - m_new))
        m = m_new

    zt = jnp.dot(h16, w2t_ref[...].astype(jnp.bfloat16),
                 preferred_element_type=jnp.float32) + b2t_ref[...]
    m_new = jnp.maximum(m, jnp.max(zt))
    s = s * jnp.exp(m - m_new) + jnp.sum(jnp.exp(zt - m_new))
    lse = m_new + jnp.log(s)
    out_ref[:, pl.ds(_NCH * _BC, _TPAD)] = zt - lse

    for c in range(_NCH):
        sl = pl.ds(c * _BC, _BC)
        out_ref[:, sl] = out_ref[:, sl] - lse


def kernel(inputs, emb, W1, b1, W2, b2):
    idx = jnp.asarray(inputs, jnp.int32)
    W1r = W1.reshape(2 * _CTX, _EMB, _HID)
    b1r = b1.reshape(1, _HID)
    b2r = b2.reshape(1, _VOCAB)
    w2t = jnp.pad(lax.slice(W2, (0, _NCH * _BC), (_HID, _VOCAB)),
                  ((0, 0), (0, _TPAD - _TAIL)))
    b2t = jnp.pad(lax.slice(b2r, (0, _NCH * _BC), (1, _VOCAB)),
                  ((0, 0), (0, _TPAD - _TAIL)), constant_values=-3.0e38)

    grid_spec = pltpu.PrefetchScalarGridSpec(
        num_scalar_prefetch=1,
        grid=(1,),
        in_specs=[
            pl.BlockSpec(memory_space=pltpu.HBM),
            pl.BlockSpec((2 * _CTX, _EMB, _HID), lambda i, idx_ref: (0, 0, 0)),
            pl.BlockSpec((1, _HID), lambda i, idx_ref: (0, 0)),
            pl.BlockSpec(memory_space=pltpu.HBM),
            pl.BlockSpec((1, _VOCAB), lambda i, idx_ref: (0, 0)),
            pl.BlockSpec((_HID, _TPAD), lambda i, idx_ref: (0, 0)),
            pl.BlockSpec((1, _TPAD), lambda i, idx_ref: (0, 0)),
        ],
        out_specs=pl.BlockSpec((1, _VPAD), lambda i, idx_ref: (0, 0)),
        scratch_shapes=[
            pltpu.VMEM((2 * _CTX, _EMB), jnp.float32),
            pltpu.VMEM((_NBUF, _HID, _BC), jnp.float32),
            pltpu.SemaphoreType.DMA((_NBUF,)),
            pltpu.SemaphoreType.DMA,
        ],
    )

    out = pl.pallas_call(
        _body,
        grid_spec=grid_spec,
        out_shape=jax.ShapeDtypeStruct((1, _VPAD), jnp.float32),
    )(idx, emb, W1r, b1r, W2, b2r, w2t, b2t)
    return out[:, :_VOCAB]
